# baseline (speedup 1.0000x reference)
import functools

import jax
import jax.numpy as jnp
from jax import lax
from jax.experimental import pallas as pl
from jax.experimental.pallas import tpu as pltpu
from jax.experimental.pallas import tpu_sc as plsc

E = 8
K = 2
D = 1024
H = 2048
O = 1024
N = 2048
P = N * K
BM = 512
NT = P // BM + E - 1
BG = 512


def _gate_block(x_ref, wg_ref, bg_ref, eid_ref, cw_ref):
    xb = x_ref[...]
    logits = lax.dot_general(
        xb, wg_ref[...], (((1,), (1,)), ((), ())),
        preferred_element_type=jnp.float32) + bg_ref[...]
    colid = lax.broadcasted_iota(jnp.int32, (BG, E), 1)
    v1 = jnp.max(logits, axis=1, keepdims=True)
    i1 = jnp.min(jnp.where(logits == v1, colid, E), axis=1, keepdims=True)
    masked = jnp.where(colid == i1, -jnp.inf, logits)
    v2 = jnp.max(masked, axis=1, keepdims=True)
    i2 = jnp.min(jnp.where(masked == v2, colid, E), axis=1, keepdims=True)
    t = jnp.exp(v2 - v1)
    w1 = 1.0 / (1.0 + t)
    w2 = 1.0 - w1
    eid_ref[...] = jnp.concatenate([i1, i2], axis=1)
    cw_ref[...] = jnp.concatenate([w1, w2], axis=1)


def _gate(x, Wg, bg):
    return pl.pallas_call(
        _gate_block,
        grid=(N // BG,),
        in_specs=[
            pl.BlockSpec((BG, D), lambda i: (i, 0)),
            pl.BlockSpec((E, D), lambda i: (0, 0)),
            pl.BlockSpec((1, E), lambda i: (0, 0)),
        ],
        out_specs=[
            pl.BlockSpec((BG, K), lambda i: (i, 0)),
            pl.BlockSpec((BG, K), lambda i: (i, 0)),
        ],
        out_shape=[
            jax.ShapeDtypeStruct((N, K), jnp.int32),
            jax.ShapeDtypeStruct((N, K), jnp.float32),
        ],
    )(x, Wg, bg.reshape(1, E))


def _ffn_block(g_ref, m_ref, lo_ref, hi_ref, first_ref,
               xs_ref, cw_ref, w1_ref, b1_ref, w2_ref, b2_ref, out_ref):
    t = pl.program_id(0)
    lo = lo_ref[t]
    hi = hi_ref[t]
    first = first_ref[t]
    m = m_ref[t]

    @pl.when(hi > lo)
    def _compute():
        xb = xs_ref[...]
        h = lax.dot_general(
            xb, w1_ref[0], (((1,), (1,)), ((), ())),
            preferred_element_type=jnp.float32) + b1_ref[0]
        h = jnp.maximum(h, 0.0)
        y = lax.dot_general(
            h, w2_ref[0], (((1,), (1,)), ((), ())),
            preferred_element_type=jnp.float32) + b2_ref[0]
        row = m * BM + lax.broadcasted_iota(jnp.int32, (BM, 1), 0)
        mask = (row >= lo) & (row < hi)
        val = jnp.where(mask, cw_ref[...] * y, 0.0)

        @pl.when(first == 1)
        def _init():
            out_ref[...] = val

        @pl.when(first == 0)
        def _acc():
            out_ref[...] += val


def _ffn(xs, cws, W1, b1, W2, b2, g, m, lo, hi, first):
    grid_spec = pltpu.PrefetchScalarGridSpec(
        num_scalar_prefetch=5,
        grid=(NT,),
        in_specs=[
            pl.BlockSpec((BM, D), lambda t, g, m, lo, hi, fs: (m[t], 0)),
            pl.BlockSpec((BM, 1), lambda t, g, m, lo, hi, fs: (m[t], 0)),
            pl.BlockSpec((1, H, D), lambda t, g, m, lo, hi, fs: (g[t], 0, 0)),
            pl.BlockSpec((1, 1, H), lambda t, g, m, lo, hi, fs: (g[t], 0, 0)),
            pl.BlockSpec((1, O, H), lambda t, g, m, lo, hi, fs: (g[t], 0, 0)),
            pl.BlockSpec((1, 1, O), lambda t, g, m, lo, hi, fs: (g[t], 0, 0)),
        ],
        out_specs=pl.BlockSpec((BM, O), lambda t, g, m, lo, hi, fs: (m[t], 0)),
    )
    return pl.pallas_call(
        _ffn_block,
        grid_spec=grid_spec,
        out_shape=jax.ShapeDtypeStruct((P, O), jnp.float32),
        compiler_params=pltpu.CompilerParams(
            dimension_semantics=("arbitrary",)),
    )(g, m, lo, hi, first, xs, cws.reshape(P, 1),
      W1, b1.reshape(E, 1, H), W2, b2.reshape(E, 1, O))


def _metadata(counts):
    c = counts[:E].astype(jnp.int32)
    s = jnp.cumsum(c) - c
    end = s + c
    first_tile = s // BM
    ntiles = jnp.where(c > 0, (end - 1) // BM - first_tile + 1, 0)
    a_total = jnp.sum(ntiles)
    cumt = jnp.cumsum(ntiles) - ntiles
    g = jnp.repeat(jnp.arange(E, dtype=jnp.int32), ntiles,
                   total_repeat_length=NT)
    tix = jnp.arange(NT, dtype=jnp.int32)
    m = first_tile[g] + (tix - cumt[g])
    active = tix < a_total
    m = jnp.minimum(jnp.where(active, m, P // BM - 1), P // BM - 1)
    lo = jnp.where(active, jnp.maximum(s[g], m * BM), 0)
    hi = jnp.where(active, jnp.minimum(end[g], (m + 1) * BM), 0)
    prev_m = jnp.concatenate([jnp.full((1,), -1, jnp.int32), m[:-1]])
    first = (active & (m != prev_m)).astype(jnp.int32)
    return g, m, lo, hi, first


@jax.jit
def kernel(x, Wg, bg, W1, b1, W2, b2):
    eid2, cw2 = _gate(x, Wg, bg)
    eidp = eid2.reshape(P)
    cwp = cw2.reshape(P)

    order = jnp.argsort(eidp, stable=True)
    pos = jnp.argsort(order)
    tok_sorted = order // K
    cw_sorted = cwp[order]
    counts = jnp.sum(eidp[:, None] == jnp.arange(E)[None, :], axis=0,
                     dtype=jnp.int32)

    g, m, lo, hi, first = _metadata(counts)

    xs = x[tok_sorted]

    ysw = _ffn(xs, cw_sorted, W1, b1, W2, b2, g, m, lo, hi, first)

    out = ysw[pos[0::K]] + ysw[pos[1::K]]
    return out
